# baseline (speedup 1.0000x reference)
import functools

import jax
import jax.numpy as jnp
from jax.experimental import pallas as pl
from jax.experimental.pallas import tpu as pltpu

_H, _W = 128, 128


def _scatter_body(starts_ref, perm_ref, cells_ref, x_ref, out_ref, xs_ref,
                  *, M: int, N: int, K2: int, nsub: int, Wc: int):
    b = pl.program_id(0)
    p2 = (perm_ref[0] == jax.lax.broadcasted_iota(jnp.int32, (M, M), 1))
    xs_ref[...] = jax.lax.dot_general(
        p2.astype(jnp.float32), x_ref[0], (((1,), (0,)), ((), ())),
        preferred_element_type=jnp.float32)

    for j2 in range(nsub):
        lo = starts_ref[b, j2]
        hi = starts_ref[b, j2 + 1]
        nch = (hi - lo + (Wc - 1)) // Wc
        base = j2 * K2

        def chunk(c, acc):
            st = jnp.minimum(lo + c * Wc, M - Wc)
            cells = cells_ref[0, pl.ds(st, Wc), :]
            onehot = (cells == jax.lax.broadcasted_iota(
                jnp.int32, (Wc, K2), 1) + base).astype(jnp.float32)
            xc = xs_ref[pl.ds(st, Wc), :]
            return acc + jax.lax.dot_general(
                xc, onehot, (((0,), (0,)), ((), ())),
                preferred_element_type=jnp.float32)

        acc = jax.lax.fori_loop(
            0, nch, chunk, jnp.zeros((N, K2), jnp.float32))
        out_ref[0, :, base:base + K2] = acc


def kernel(x, spatial_size, location):
    B, M, N = x.shape
    H, W = _H, _W
    HW = H * W
    index = (location[:, :, 0] * spatial_size[1] + location[:, :, 1]) % HW

    cells_sorted = jnp.sort(index, axis=1)
    perm = jnp.argsort(index, axis=1).astype(jnp.int32)

    K2 = 512
    nsub = HW // K2
    Wc = 64
    edges = jnp.arange(nsub + 1, dtype=jnp.int32) * K2
    starts = jax.vmap(
        lambda sc: jnp.searchsorted(sc, edges).astype(jnp.int32)
    )(cells_sorted)

    grid_spec = pltpu.PrefetchScalarGridSpec(
        num_scalar_prefetch=1,
        grid=(B,),
        in_specs=[
            pl.BlockSpec((1, M, 1), lambda b, *_: (b, 0, 0)),
            pl.BlockSpec((1, M, 1), lambda b, *_: (b, 0, 0)),
            pl.BlockSpec((1, M, N), lambda b, *_: (b, 0, 0)),
        ],
        out_specs=pl.BlockSpec((1, N, HW), lambda b, *_: (b, 0, 0)),
        scratch_shapes=[pltpu.VMEM((M, N), jnp.float32)],
    )
    out = pl.pallas_call(
        functools.partial(
            _scatter_body, M=M, N=N, K2=K2, nsub=nsub, Wc=Wc),
        grid_spec=grid_spec,
        out_shape=jax.ShapeDtypeStruct((B, N, HW), jnp.float32),
    )(starts, perm.reshape(B, M, 1), cells_sorted.reshape(B, M, 1), x)
    return out.reshape(B, N, H, W)


# speedup vs baseline: 2.0542x; 2.0542x over previous
import functools

import jax
import jax.numpy as jnp
from jax.experimental import pallas as pl

_H, _W = 128, 128


def _scatter_body(idx_ref, xt_ref, out_ref, *, M: int, K2: int, nsub: int):
    idx = idx_ref[0, 0, :]
    for j2 in range(nsub):
        cols = jax.lax.broadcasted_iota(jnp.int32, (M, K2), 1) + j2 * K2
        onehot = (idx[:, None] == cols).astype(jnp.float32)
        out_ref[0, :, j2 * K2:(j2 + 1) * K2] = jax.lax.dot_general(
            xt_ref[0], onehot, (((1,), (0,)), ((), ())),
            preferred_element_type=jnp.float32)


def kernel(x, spatial_size, location):
    B, M, N = x.shape
    H, W = _H, _W
    HW = H * W
    index = (location[:, :, 0] * spatial_size[1] + location[:, :, 1]) % HW
    index = index.reshape(B, 1, M)
    xt = jnp.transpose(x, (0, 2, 1))

    K2 = 1024
    nsub = HW // K2
    out = pl.pallas_call(
        functools.partial(_scatter_body, M=M, K2=K2, nsub=nsub),
        grid=(B,),
        in_specs=[
            pl.BlockSpec((1, 1, M), lambda b: (b, 0, 0)),
            pl.BlockSpec((1, N, M), lambda b: (b, 0, 0)),
        ],
        out_specs=pl.BlockSpec((1, N, HW), lambda b: (b, 0, 0)),
        out_shape=jax.ShapeDtypeStruct((B, N, HW), jnp.float32),
    )(index, xt)
    return out.reshape(B, N, H, W)
